# speedup vs baseline: 2.4579x; 2.4579x over previous
import functools

import jax
import jax.numpy as jnp
from jax import lax
from jax.experimental import pallas as pl
from jax.experimental.pallas import tpu as pltpu
from jax.experimental.pallas import tpu_sc as plsc

B = 1024
L = 200
H = 128
N_TOK = B * L
EPS = 1e-12

NC = 2
NS = 16
NW = NC * NS
TOK_PER_W = N_TOK // NW
CHUNK = 128
N_CHUNKS = TOK_PER_W // CHUNK


def _sc_gather(table, ids_flat):
    mesh = plsc.VectorSubcoreMesh(
        core_axis_name="c", subcore_axis_name="s", num_cores=NC, num_subcores=NS
    )

    @functools.partial(
        pl.kernel,
        mesh=mesh,
        out_type=jax.ShapeDtypeStruct((N_TOK, H), jnp.float32),
        scratch_types=[
            pltpu.VMEM((CHUNK,), jnp.int32),
            pltpu.VMEM((CHUNK, H), jnp.float32),
            pltpu.SemaphoreType.DMA,
        ],
    )
    def k(table_hbm, idx_hbm, out_hbm, idx_v, rows_v, sem):
        wid = lax.axis_index("s") * NC + lax.axis_index("c")
        base = wid * TOK_PER_W

        def body(i, carry):
            off = base + i * CHUNK
            pltpu.sync_copy(idx_hbm.at[pl.ds(off, CHUNK)], idx_v)
            pltpu.async_copy(table_hbm.at[idx_v], rows_v, sem).wait()
            pltpu.sync_copy(rows_v, out_hbm.at[pl.ds(off, CHUNK)])
            return carry

        lax.fori_loop(0, N_CHUNKS, body, 0)

    return k(table, ids_flat)


TBLK = 1024


def _tc_body(g_ref, tt_ref, age_ref, ap_ref, seg_ref, wa_ref, pa_ref,
             wb_ref, pb_ref, gam_ref, bet_ref, o_ref):
    x = g_ref[...]
    seg = jnp.where(tt_ref[...] == 0, seg_ref[0:1, :], seg_ref[1:2, :])
    lane = lax.broadcasted_iota(jnp.int32, (1, H), 1)
    arg_a = age_ref[...] * wa_ref[...] + pa_ref[...]
    t2v_a = jnp.where(lane == 0, arg_a, jnp.cos(arg_a))
    arg_b = ap_ref[...] * wb_ref[...] + pb_ref[...]
    t2v_b = jnp.where(lane == 0, arg_b, jnp.cos(arg_b))
    x = x + seg + t2v_a + t2v_b
    mu = jnp.mean(x, axis=-1, keepdims=True)
    xc = x - mu
    var = jnp.mean(xc * xc, axis=-1, keepdims=True)
    o_ref[...] = xc * lax.rsqrt(var + EPS) * gam_ref[...] + bet_ref[...]


def _tc_fuse(gathered, tt, age_c, ap_c, seg_tbl, wa, pa, wb, pb, gamma, beta):
    grid = (N_TOK // TBLK,)
    tok_spec = pl.BlockSpec((TBLK, H), lambda i: (i, 0))
    col_spec = pl.BlockSpec((TBLK, 1), lambda i: (i, 0))

    def rep(shape):
        return pl.BlockSpec(shape, lambda i: (0, 0))

    return pl.pallas_call(
        _tc_body,
        grid=grid,
        in_specs=[
            tok_spec, col_spec, col_spec, col_spec,
            rep((2, H)), rep((1, H)), rep((1, H)), rep((1, H)), rep((1, H)),
            rep((1, H)), rep((1, H)),
        ],
        out_specs=tok_spec,
        out_shape=jax.ShapeDtypeStruct((N_TOK, H), jnp.float32),
    )(gathered, tt, age_c, ap_c, seg_tbl, wa, pa, wb, pb, gamma, beta)


def kernel(input_ids, token_type_ids, age, abspos, concept_table,
           segment_table, age_w0, age_phi0, age_w, age_phi,
           abspos_w0, abspos_phi0, abspos_w, abspos_phi,
           ln_gamma, ln_beta):
    ids_flat = input_ids.reshape(-1).astype(jnp.int32)
    gathered = _sc_gather(concept_table, ids_flat)

    tt = token_type_ids.reshape(-1, 1).astype(jnp.int32)
    age_c = age.reshape(-1, 1)
    ap_c = abspos.reshape(-1, 1)
    wa = jnp.concatenate([age_w0, age_w], axis=1)
    pa = jnp.concatenate([age_phi0, age_phi])[None, :]
    wb = jnp.concatenate([abspos_w0, abspos_w], axis=1)
    pb = jnp.concatenate([abspos_phi0, abspos_phi])[None, :]

    out = _tc_fuse(gathered, tt, age_c, ap_c, segment_table, wa, pa, wb, pb,
                   ln_gamma[None, :], ln_beta[None, :])
    return out.reshape(B, L, H)


# speedup vs baseline: 4.6537x; 1.8933x over previous
import functools

import jax
import jax.numpy as jnp
from jax import lax
from jax.experimental import pallas as pl
from jax.experimental.pallas import tpu as pltpu
from jax.experimental.pallas import tpu_sc as plsc

B = 1024
L = 200
H = 128
N_TOK = B * L
EPS = 1e-12

NC = 2
NS = 16
NW = NC * NS
TOK_PER_W = N_TOK // NW
CHUNK = 128
N_CHUNKS = TOK_PER_W // CHUNK


def _sc_gather(table, ids_flat):
    mesh = plsc.VectorSubcoreMesh(
        core_axis_name="c", subcore_axis_name="s", num_cores=NC, num_subcores=NS
    )

    @functools.partial(
        pl.kernel,
        mesh=mesh,
        out_type=jax.ShapeDtypeStruct((N_TOK, H), jnp.float32),
        scratch_types=[
            pltpu.VMEM((CHUNK,), jnp.int32),
            pltpu.VMEM((CHUNK, H), jnp.float32),
            pltpu.SemaphoreType.DMA,
        ],
    )
    def k(table_hbm, idx_hbm, out_hbm, idx_v, rows_v, sem):
        wid = lax.axis_index("s") * NC + lax.axis_index("c")
        base = wid * TOK_PER_W

        def body(i, carry):
            off = base + i * CHUNK
            pltpu.sync_copy(idx_hbm.at[pl.ds(off, CHUNK)], idx_v)
            pltpu.async_copy(table_hbm.at[idx_v], rows_v, sem).wait()
            pltpu.sync_copy(rows_v, out_hbm.at[pl.ds(off, CHUNK)])
            return carry

        lax.fori_loop(0, N_CHUNKS, body, 0)

    return k(table, ids_flat)


TBLK = 1024

_INV2PI = 0.15915494309189535
_CW1 = 6.28125
_CW2 = 0.0019350051879882812
_CW3 = 3.019916050561733e-07
_COS_C = (0.9999999908225348, -0.4999999049565284, 0.041666507182362957,
          -0.0013887887278042725, 2.47716297068164e-05,
          -2.70957056196639e-07, 1.7304459439821245e-09)


def _fast_cos(arg):
    q = arg * _INV2PI
    kf = lax.floor(q + 0.5)
    r = ((arg - kf * _CW1) - kf * _CW2) - kf * _CW3
    s = r * r
    p = jnp.float32(_COS_C[6])
    for i in (5, 4, 3, 2, 1, 0):
        p = p * s + jnp.float32(_COS_C[i])
    return p


def _tc_body(g_ref, tt_ref, age_ref, ap_ref, seg_ref, wa_ref, pa_ref,
             wb_ref, pb_ref, gam_ref, bet_ref, o_ref):
    x = g_ref[...]
    seg = jnp.where(tt_ref[...] == 0, seg_ref[0:1, :], seg_ref[1:2, :])
    lane = lax.broadcasted_iota(jnp.int32, (1, H), 1)
    arg_a = age_ref[...] * wa_ref[...] + pa_ref[...]
    t2v_a = jnp.where(lane == 0, arg_a, _fast_cos(arg_a))
    arg_b = ap_ref[...] * wb_ref[...] + pb_ref[...]
    t2v_b = jnp.where(lane == 0, arg_b, _fast_cos(arg_b))
    x = x + seg + t2v_a + t2v_b
    mu = jnp.mean(x, axis=-1, keepdims=True)
    xc = x - mu
    var = jnp.mean(xc * xc, axis=-1, keepdims=True)
    o_ref[...] = xc * lax.rsqrt(var + EPS) * gam_ref[...] + bet_ref[...]


def _tc_fuse(gathered, tt, age_c, ap_c, seg_tbl, wa, pa, wb, pb, gamma, beta):
    grid = (N_TOK // TBLK,)
    tok_spec = pl.BlockSpec((TBLK, H), lambda i: (i, 0))
    col_spec = pl.BlockSpec((TBLK, 1), lambda i: (i, 0))

    def rep(shape):
        return pl.BlockSpec(shape, lambda i: (0, 0))

    return pl.pallas_call(
        _tc_body,
        grid=grid,
        in_specs=[
            tok_spec, col_spec, col_spec, col_spec,
            rep((2, H)), rep((1, H)), rep((1, H)), rep((1, H)), rep((1, H)),
            rep((1, H)), rep((1, H)),
        ],
        out_specs=tok_spec,
        out_shape=jax.ShapeDtypeStruct((N_TOK, H), jnp.float32),
    )(gathered, tt, age_c, ap_c, seg_tbl, wa, pa, wb, pb, gamma, beta)


def kernel(input_ids, token_type_ids, age, abspos, concept_table,
           segment_table, age_w0, age_phi0, age_w, age_phi,
           abspos_w0, abspos_phi0, abspos_w, abspos_phi,
           ln_gamma, ln_beta):
    ids_flat = input_ids.reshape(-1).astype(jnp.int32)
    gathered = _sc_gather(concept_table, ids_flat)

    tt = token_type_ids.reshape(-1, 1).astype(jnp.int32)
    age_c = age.reshape(-1, 1)
    ap_c = abspos.reshape(-1, 1)
    wa = jnp.concatenate([age_w0, age_w], axis=1)
    pa = jnp.concatenate([age_phi0, age_phi])[None, :]
    wb = jnp.concatenate([abspos_w0, abspos_w], axis=1)
    pb = jnp.concatenate([abspos_phi0, abspos_phi])[None, :]

    out = _tc_fuse(gathered, tt, age_c, ap_c, segment_table, wa, pa, wb, pb,
                   ln_gamma[None, :], ln_beta[None, :])
    return out.reshape(B, L, H)
